# baseline (speedup 1.0000x reference)
import functools

import jax
import jax.numpy as jnp
from jax import lax
from jax.experimental import pallas as pl
from jax.experimental.pallas import tpu as pltpu
from jax.experimental.pallas import tpu_sc as plsc

N = 10000
E = 320000
D = 128

NC = 2
NS = 16
NW = NC * NS

CH = 128
NCH = 80
NCH2 = NCH // 2
E_PAD = NW * NCH * CH

NP = 10240
ROWS_PER_TILE = NP // NS


def _sc_segment_sum(support, src2d, dst2d, zeros):
  mesh = plsc.VectorSubcoreMesh(core_axis_name="c", subcore_axis_name="s")

  @functools.partial(
      pl.kernel,
      out_type=(
          jax.ShapeDtypeStruct((NP, D), jnp.float32),
          jax.ShapeDtypeStruct((NP, D), jnp.float32),
      ),
      mesh=mesh,
      scratch_types=[
          pltpu.VMEM((CH,), jnp.int32),
          pltpu.VMEM((CH,), jnp.int32),
          pltpu.VMEM((CH,), jnp.int32),
          pltpu.VMEM((CH,), jnp.int32),
          pltpu.VMEM((CH, D), jnp.float32),
          pltpu.VMEM((CH, D), jnp.float32),
          pltpu.VMEM_SHARED((NP, D), jnp.float32),
          pltpu.SemaphoreType.DMA,
          pltpu.SemaphoreType.DMA,
      ],
  )
  def k(support_hbm, src_hbm, dst_hbm, zeros_hbm, p0_hbm, p1_hbm,
        src0, dst0, src1, dst1, rows0, rows1, acc, sem0, sem1):
    cid = lax.axis_index("c")
    sid = lax.axis_index("s")
    wid = cid * NS + sid

    r0 = sid * ROWS_PER_TILE
    pltpu.sync_copy(zeros_hbm.at[pl.ds(r0, ROWS_PER_TILE)],
                    acc.at[pl.ds(r0, ROWS_PER_TILE)])

    plsc.subcore_barrier()

    ebase = wid * NCH * CH

    def load_idx(ci, src_v, dst_v):
      eoff = pl.multiple_of(ebase + ci * CH, 8)
      pltpu.sync_copy(src_hbm.at[pl.ds(eoff, CH)], src_v)
      pltpu.sync_copy(dst_hbm.at[pl.ds(eoff, CH)], dst_v)

    def gather(src_v, buf, sem):
      pltpu.async_copy(support_hbm.at[src_v], buf, sem)

    def wait_gather(buf, sem):
      pltpu.make_async_copy(zeros_hbm.at[pl.ds(0, CH)], buf, sem).wait()

    def scatter(dst_v, buf):
      pltpu.sync_copy(buf, acc.at[dst_v], add=True)

    load_idx(0, src0, dst0)
    gather(src0, rows0, sem0)
    load_idx(1, src1, dst1)
    gather(src1, rows1, sem1)

    def body(c, carry):
      c2 = 2 * c
      wait_gather(rows0, sem0)
      scatter(dst0, rows0)
      load_idx(c2 + 2, src0, dst0)
      gather(src0, rows0, sem0)
      wait_gather(rows1, sem1)
      scatter(dst1, rows1)
      load_idx(c2 + 3, src1, dst1)
      gather(src1, rows1, sem1)
      return carry

    lax.fori_loop(0, NCH // 2 - 1, body, 0)

    wait_gather(rows0, sem0)
    scatter(dst0, rows0)
    wait_gather(rows1, sem1)
    scatter(dst1, rows1)

    plsc.subcore_barrier()

    @pl.when(cid == 0)
    def _():
      pltpu.sync_copy(acc.at[pl.ds(r0, ROWS_PER_TILE)],
                      p0_hbm.at[pl.ds(r0, ROWS_PER_TILE)])

    @pl.when(cid == 1)
    def _():
      pltpu.sync_copy(acc.at[pl.ds(r0, ROWS_PER_TILE)],
                      p1_hbm.at[pl.ds(r0, ROWS_PER_TILE)])

  return k(support, src2d, dst2d, zeros)


_BLK = 1000


def _tc_matmul(x, w, b):
  def body(x_ref, w_ref, b_ref, o_ref):
    o_ref[...] = jnp.dot(x_ref[...], w_ref[...],
                         preferred_element_type=jnp.float32) + b_ref[...]

  return pl.pallas_call(
      body,
      grid=(N // _BLK,),
      in_specs=[
          pl.BlockSpec((_BLK, D), lambda i: (i, 0)),
          pl.BlockSpec((D, D), lambda i: (0, 0)),
          pl.BlockSpec((1, D), lambda i: (0, 0)),
      ],
      out_specs=pl.BlockSpec((_BLK, D), lambda i: (i, 0)),
      out_shape=jax.ShapeDtypeStruct((N, D), jnp.float32),
  )(x, w, b)


def _tc_relu_add_matmul(p0, p1, w, b):
  def body(p0_ref, p1_ref, w_ref, b_ref, o_ref):
    h = jnp.maximum(p0_ref[...] + p1_ref[...], 0.0)
    o_ref[...] = jnp.dot(h, w_ref[...],
                         preferred_element_type=jnp.float32) + b_ref[...]

  return pl.pallas_call(
      body,
      grid=(N // _BLK,),
      in_specs=[
          pl.BlockSpec((_BLK, D), lambda i: (i, 0)),
          pl.BlockSpec((_BLK, D), lambda i: (i, 0)),
          pl.BlockSpec((D, D), lambda i: (0, 0)),
          pl.BlockSpec((1, D), lambda i: (0, 0)),
      ],
      out_specs=pl.BlockSpec((_BLK, D), lambda i: (i, 0)),
      out_shape=jax.ShapeDtypeStruct((N, D), jnp.float32),
  )(p0, p1, w, b)


def _tc_add(p0, p1):
  def body(p0_ref, p1_ref, o_ref):
    o_ref[...] = p0_ref[...] + p1_ref[...]

  return pl.pallas_call(
      body,
      grid=(N // _BLK,),
      in_specs=[
          pl.BlockSpec((_BLK, D), lambda i: (i, 0)),
          pl.BlockSpec((_BLK, D), lambda i: (i, 0)),
      ],
      out_specs=pl.BlockSpec((_BLK, D), lambda i: (i, 0)),
      out_shape=jax.ShapeDtypeStruct((N, D), jnp.float32),
  )(p0, p1)


def _pad_edges(edge_index):
  pad = E_PAD - E
  src = jnp.concatenate([edge_index[0], jnp.zeros((pad,), jnp.int32)])
  dst = jnp.concatenate([edge_index[1], jnp.full((pad,), N, jnp.int32)])
  return src, dst


@jax.jit
def kernel(x, adj, W1, b1, W2, b2):
  adj = adj.astype(jnp.int32)
  src1, dst1 = _pad_edges(adj[0])
  src2, dst2 = _pad_edges(adj[1])
  zeros = jnp.zeros((NP, D), jnp.float32)
  b1r = b1.reshape(1, D)
  b2r = b2.reshape(1, D)

  support1 = _tc_matmul(x, W1, b1r)
  p0, p1 = _sc_segment_sum(support1, src1, dst1, zeros)
  support2 = _tc_relu_add_matmul(p0, p1, W2, b2r)
  q0, q1 = _sc_segment_sum(support2, src2, dst2, zeros)
  return _tc_add(q0, q1)


# speedup vs baseline: 1.0056x; 1.0056x over previous
import functools

import jax
import jax.numpy as jnp
from jax import lax
from jax.experimental import pallas as pl
from jax.experimental.pallas import tpu as pltpu
from jax.experimental.pallas import tpu_sc as plsc

N = 10000
E = 320000
D = 128

NC = 2
NS = 16
NW = NC * NS

CH = 128
NCH = 80
NCH2 = NCH // 2
NBUF = 2
E_PAD = NW * NCH * CH

NP = 10240
ROWS_PER_TILE = NP // NS


def _sc_segment_sum(support, src2d, dst2d, zeros):
  mesh = plsc.VectorSubcoreMesh(core_axis_name="c", subcore_axis_name="s")

  @functools.partial(
      pl.kernel,
      out_type=(
          jax.ShapeDtypeStruct((NP, D), jnp.float32),
          jax.ShapeDtypeStruct((NP, D), jnp.float32),
      ),
      mesh=mesh,
      scratch_types=[
          pltpu.VMEM((NCH2, CH), jnp.int32),
          pltpu.VMEM((NCH2, CH), jnp.int32),
          pltpu.VMEM((NBUF, CH, D), jnp.float32),
          pltpu.VMEM_SHARED((NP, D), jnp.float32),
          pltpu.SemaphoreType.DMA((NBUF,)),
      ],
  )
  def k(support_hbm, src_hbm, dst_hbm, zeros_hbm, p0_hbm, p1_hbm,
        src_h, dst_h, rows, acc, sems):
    cid = lax.axis_index("c")
    sid = lax.axis_index("s")
    wid = cid * NS + sid

    r0 = sid * ROWS_PER_TILE
    pltpu.sync_copy(zeros_hbm.at[pl.ds(r0, ROWS_PER_TILE)],
                    acc.at[pl.ds(r0, ROWS_PER_TILE)])

    def load_half(h):
      rbase = pl.multiple_of(wid * NCH + h * NCH2, 8)
      pltpu.sync_copy(src_hbm.at[pl.ds(rbase, NCH2)], src_h)
      pltpu.sync_copy(dst_hbm.at[pl.ds(rbase, NCH2)], dst_h)

    def gather(c, j):
      pltpu.async_copy(support_hbm.at[src_h.at[c]], rows.at[j], sems.at[j])

    def wait_gather(j):
      pltpu.make_async_copy(zeros_hbm.at[pl.ds(0, CH)], rows.at[j],
                            sems.at[j]).wait()

    def scatter(c, j):
      pltpu.sync_copy(rows.at[j], acc.at[dst_h.at[c]], add=True)

    load_half(0)
    plsc.subcore_barrier()

    def body(i, carry):
      c = i * NBUF
      for j in range(NBUF):
        wait_gather(j)
        scatter(c + j, j)
        gather(c + NBUF + j, j)
      return carry

    for h in range(2):
      if h:
        load_half(1)
      for j in range(NBUF):
        gather(j, j)
      lax.fori_loop(0, NCH2 // NBUF - 1, body, 0)
      for j in range(NBUF):
        wait_gather(j)
        scatter(NCH2 - NBUF + j, j)

    plsc.subcore_barrier()

    @pl.when(cid == 0)
    def _():
      pltpu.sync_copy(acc.at[pl.ds(r0, ROWS_PER_TILE)],
                      p0_hbm.at[pl.ds(r0, ROWS_PER_TILE)])

    @pl.when(cid == 1)
    def _():
      pltpu.sync_copy(acc.at[pl.ds(r0, ROWS_PER_TILE)],
                      p1_hbm.at[pl.ds(r0, ROWS_PER_TILE)])

  return k(support, src2d, dst2d, zeros)


_BLK = 1000


def _tc_matmul(x, w, b):
  def body(x_ref, w_ref, b_ref, o_ref):
    o_ref[...] = jnp.dot(x_ref[...], w_ref[...],
                         preferred_element_type=jnp.float32) + b_ref[...]

  return pl.pallas_call(
      body,
      grid=(N // _BLK,),
      in_specs=[
          pl.BlockSpec((_BLK, D), lambda i: (i, 0)),
          pl.BlockSpec((D, D), lambda i: (0, 0)),
          pl.BlockSpec((1, D), lambda i: (0, 0)),
      ],
      out_specs=pl.BlockSpec((_BLK, D), lambda i: (i, 0)),
      out_shape=jax.ShapeDtypeStruct((N, D), jnp.float32),
  )(x, w, b)


def _tc_relu_add_matmul(p0, p1, w, b):
  def body(p0_ref, p1_ref, w_ref, b_ref, o_ref):
    h = jnp.maximum(p0_ref[...] + p1_ref[...], 0.0)
    o_ref[...] = jnp.dot(h, w_ref[...],
                         preferred_element_type=jnp.float32) + b_ref[...]

  return pl.pallas_call(
      body,
      grid=(N // _BLK,),
      in_specs=[
          pl.BlockSpec((_BLK, D), lambda i: (i, 0)),
          pl.BlockSpec((_BLK, D), lambda i: (i, 0)),
          pl.BlockSpec((D, D), lambda i: (0, 0)),
          pl.BlockSpec((1, D), lambda i: (0, 0)),
      ],
      out_specs=pl.BlockSpec((_BLK, D), lambda i: (i, 0)),
      out_shape=jax.ShapeDtypeStruct((N, D), jnp.float32),
  )(p0, p1, w, b)


def _tc_add(p0, p1):
  def body(p0_ref, p1_ref, o_ref):
    o_ref[...] = p0_ref[...] + p1_ref[...]

  return pl.pallas_call(
      body,
      grid=(N // _BLK,),
      in_specs=[
          pl.BlockSpec((_BLK, D), lambda i: (i, 0)),
          pl.BlockSpec((_BLK, D), lambda i: (i, 0)),
      ],
      out_specs=pl.BlockSpec((_BLK, D), lambda i: (i, 0)),
      out_shape=jax.ShapeDtypeStruct((N, D), jnp.float32),
  )(p0, p1)


def _pad_edges(edge_index):
  pad = E_PAD - E
  src = jnp.concatenate([edge_index[0], jnp.zeros((pad,), jnp.int32)])
  dst = jnp.concatenate([edge_index[1], jnp.full((pad,), N, jnp.int32)])
  return src.reshape(NW * NCH, CH), dst.reshape(NW * NCH, CH)


@jax.jit
def kernel(x, adj, W1, b1, W2, b2):
  adj = adj.astype(jnp.int32)
  src1, dst1 = _pad_edges(adj[0])
  src2, dst2 = _pad_edges(adj[1])
  zeros = jnp.zeros((NP, D), jnp.float32)
  b1r = b1.reshape(1, D)
  b2r = b2.reshape(1, D)

  support1 = _tc_matmul(x, W1, b1r)
  p0, p1 = _sc_segment_sum(support1, src1, dst1, zeros)
  support2 = _tc_relu_add_matmul(p0, p1, W2, b2r)
  q0, q1 = _sc_segment_sum(support2, src2, dst2, zeros)
  return _tc_add(q0, q1)
